# baseline (speedup 1.0000x reference)
import dataclasses
import functools

import jax
import jax.numpy as jnp
from jax import lax
from jax.experimental import pallas as pl
from jax.experimental.pallas import tpu as pltpu
from jax.experimental.pallas import tpu_sc as plsc

B, C, H, W = 2, 96, 384, 384
HW = H * W
NPIX = B * HW
NC, NS, L = 2, 16, 16
ROWS_W = H // NS
P = 128
CPR = W // P
NCHUNK = ROWS_W * CPR
G = P // L


def _floor(v):
  t = v.astype(jnp.int32)
  tf = t.astype(jnp.float32)
  adj = jnp.where(tf > v, 1, 0)
  return t - adj, tf - adj.astype(jnp.float32)


def _sc_resample(xt, warp):
  mesh = plsc.VectorSubcoreMesh(core_axis_name="c", subcore_axis_name="s")
  cp = pltpu.CompilerParams()
  if "needs_layout_passes" in pltpu.CompilerParams.__dataclass_fields__:
    cp = dataclasses.replace(cp, needs_layout_passes=False)
  if "use_tc_tiling_on_sc" in pltpu.CompilerParams.__dataclass_fields__:
    cp = dataclasses.replace(cp, use_tc_tiling_on_sc=False)

  @functools.partial(
      pl.kernel,
      compiler_params=cp,
      out_type=jax.ShapeDtypeStruct((B, C, H, W), jnp.float32),
      mesh=mesh,
      scratch_types=[
          [[pltpu.VMEM((P,), jnp.int32) for _ in range(4)] for _ in range(2)],
          [[pltpu.VMEM((P,), jnp.float32) for _ in range(4)] for _ in range(2)],
          [[pltpu.VMEM((P, C), jnp.float32) for _ in range(4)]
           for _ in range(2)],
          [pltpu.VMEM((P,), jnp.float32) for _ in range(2)],
          pltpu.VMEM((C, P), jnp.float32),
          [pltpu.SemaphoreType.DMA for _ in range(2)],
          pltpu.SemaphoreType.DMA,
      ],
  )
  def k(xt_hbm, warp_hbm, out_hbm, idx_vs, w_vs, row_vs, wp_vs, out_v, sems,
        osem):
    b = lax.axis_index("c")
    s = lax.axis_index("s")
    bb = b * HW
    woff0 = 2 * bb
    woff1 = 2 * bb + HW

    def stage(row, col0, st):
      q = row * W + col0
      pltpu.sync_copy(warp_hbm.at[pl.ds(woff0 + q, P)], wp_vs[0])
      pltpu.sync_copy(warp_hbm.at[pl.ds(woff1 + q, P)], wp_vs[1])
      rowf = row.astype(jnp.float32)
      for g in range(G):
        colf = (col0 + g * L).astype(jnp.float32)
        ii = lax.iota(jnp.int32, L).astype(jnp.float32)
        sl = pl.ds(g * L, L)
        sx = colf + ii + wp_vs[0][sl]
        sy = rowf + wp_vs[1][sl]
        x0i, x0f = _floor(sx)
        y0i, y0f = _floor(sy)
        wx = sx - x0f
        wy = sy - y0f
        bx0 = (x0f >= 0.0) & (x0f <= W - 1.0)
        bx1 = (x0f >= -1.0) & (x0f <= W - 2.0)
        by0 = (y0f >= 0.0) & (y0f <= H - 1.0)
        by1 = (y0f >= -1.0) & (y0f <= H - 2.0)
        ix0 = jnp.clip(x0i, 0, W - 1)
        ix1 = jnp.clip(x0i + 1, 0, W - 1)
        ry0 = bb + jnp.clip(y0i, 0, H - 1) * W
        ry1 = bb + jnp.clip(y0i + 1, 0, H - 1) * W
        idx_vs[st][0][sl] = ry0 + ix0
        idx_vs[st][1][sl] = ry0 + ix1
        idx_vs[st][2][sl] = ry1 + ix0
        idx_vs[st][3][sl] = ry1 + ix1
        zero = jnp.zeros((L,), jnp.float32)
        w_vs[st][0][sl] = jnp.where(bx0 & by0, (1.0 - wx) * (1.0 - wy), zero)
        w_vs[st][1][sl] = jnp.where(bx1 & by0, wx * (1.0 - wy), zero)
        w_vs[st][2][sl] = jnp.where(bx0 & by1, (1.0 - wx) * wy, zero)
        w_vs[st][3][sl] = jnp.where(bx1 & by1, wx * wy, zero)
      for k4 in range(4):
        pltpu.async_copy(xt_hbm.at[idx_vs[st][k4]], row_vs[st][k4], sems[st])

    def drain(st):
      for k4 in range(4):
        pltpu.make_async_copy(xt_hbm.at[idx_vs[st][k4]], row_vs[st][k4],
                              sems[st]).wait()

    def combine_out(row, col0, st, first):
      @pl.when(jnp.logical_not(first))
      def _():
        pltpu.make_async_copy(out_v, out_hbm.at[b, :, row, pl.ds(col0, P)],
                              osem).wait()

      for g in range(G):
        sl = pl.ds(g * L, L)
        w0 = w_vs[st][0][sl]
        w1 = w_vs[st][1][sl]
        w2 = w_vs[st][2][sl]
        w3 = w_vs[st][3][sl]
        pix = lax.iota(jnp.int32, L) + g * L

        @plsc.parallel_loop(0, C, 1, unroll=4)
        def _chan(c):
          cc = jnp.full((L,), c, jnp.int32)
          v0 = plsc.load_gather(row_vs[st][0], [pix, cc])
          v1 = plsc.load_gather(row_vs[st][1], [pix, cc])
          v2 = plsc.load_gather(row_vs[st][2], [pix, cc])
          v3 = plsc.load_gather(row_vs[st][3], [pix, cc])
          acc = w0 * v0 + w1 * v1 + w2 * v2 + w3 * v3
          out_v[c, pl.ds(g * L, L)] = acc

      pltpu.async_copy(out_v, out_hbm.at[b, :, row, pl.ds(col0, P)], osem)

    def rc(t):
      r3 = t // CPR
      return s * ROWS_W + r3, (t - r3 * CPR) * P

    r0, c0 = rc(jnp.int32(0))
    stage(r0, c0, 0)

    @pl.loop(0, NCHUNK, step=2)
    def _chunks(t):
      r1, c1 = rc(t + 1)
      stage(r1, c1, 1)
      ra, ca = rc(t)
      drain(0)
      combine_out(ra, ca, 0, t == 0)

      @pl.when(t + 2 < NCHUNK)
      def _():
        r2, c2 = rc(t + 2)
        stage(r2, c2, 0)

      drain(1)
      combine_out(r1, c1, 1, False)

    pltpu.make_async_copy(out_v, out_hbm.at[b, :, H - 1, pl.ds(W - P, P)],
                          osem).wait()

  return k(xt, warp)


def kernel(x, warp):
  xt = x.astype(jnp.float32).transpose(0, 2, 3, 1).reshape(NPIX, C)
  return _sc_resample(xt, warp.astype(jnp.float32).reshape(2 * NPIX))


# speedup vs baseline: 1.3146x; 1.3146x over previous
import dataclasses
import functools

import jax
import jax.numpy as jnp
from jax import lax
from jax.experimental import pallas as pl
from jax.experimental.pallas import tpu as pltpu
from jax.experimental.pallas import tpu_sc as plsc

B, C, H, W = 2, 96, 384, 384
HW = H * W
NPIX = B * HW
NC, NS, L = 2, 16, 16
ROWS_W = H // NS
P = 128
CPR = W // P
NCHUNK = ROWS_W * CPR
G = P // L
CB = C // L
BLK = 512


def _floor(v):
  t = v.astype(jnp.int32)
  tf = t.astype(jnp.float32)
  adj = jnp.where(tf > v, 1, 0)
  return t - adj, tf - adj.astype(jnp.float32)


def _tc_transpose_in(x):
  xv = x.reshape(B, C, HW)

  def body(x_ref, o_ref):
    o_ref[...] = jnp.swapaxes(x_ref[...], 1, 2)

  out = pl.pallas_call(
      body,
      grid=(B, HW // BLK),
      in_specs=[pl.BlockSpec((1, C, BLK), lambda b, i: (b, 0, i))],
      out_specs=pl.BlockSpec((1, BLK, C), lambda b, i: (b, i, 0)),
      out_shape=jax.ShapeDtypeStruct((B, HW, C), jnp.float32),
  )(xv)
  return out.reshape(NPIX, C)


def _tc_transpose_out(y):
  yv = y.reshape(B, HW, C)

  def body(y_ref, o_ref):
    o_ref[...] = jnp.swapaxes(y_ref[...], 1, 2)

  out = pl.pallas_call(
      body,
      grid=(B, HW // BLK),
      in_specs=[pl.BlockSpec((1, BLK, C), lambda b, i: (b, i, 0))],
      out_specs=pl.BlockSpec((1, C, BLK), lambda b, i: (b, 0, i)),
      out_shape=jax.ShapeDtypeStruct((B, C, HW), jnp.float32),
  )(yv)
  return out.reshape(B, C, H, W)


def _sc_resample(xt, warp):
  mesh = plsc.VectorSubcoreMesh(core_axis_name="c", subcore_axis_name="s")
  cp = pltpu.CompilerParams()
  if "needs_layout_passes" in pltpu.CompilerParams.__dataclass_fields__:
    cp = dataclasses.replace(cp, needs_layout_passes=False)
  if "use_tc_tiling_on_sc" in pltpu.CompilerParams.__dataclass_fields__:
    cp = dataclasses.replace(cp, use_tc_tiling_on_sc=False)

  @functools.partial(
      pl.kernel,
      compiler_params=cp,
      out_type=jax.ShapeDtypeStruct((NPIX, C), jnp.float32),
      mesh=mesh,
      scratch_types=[
          [[pltpu.VMEM((P,), jnp.int32) for _ in range(4)] for _ in range(2)],
          [[pltpu.VMEM((P,), jnp.float32) for _ in range(4)] for _ in range(2)],
          [[pltpu.VMEM((P, C), jnp.float32) for _ in range(4)]
           for _ in range(2)],
          [pltpu.VMEM((P,), jnp.float32) for _ in range(2)],
          pltpu.VMEM((P, C), jnp.float32),
          [pltpu.SemaphoreType.DMA for _ in range(2)],
      ],
  )
  def k(xt_hbm, warp_hbm, out_hbm, idx_vs, w_vs, row_vs, wp_vs, out_v, sems):
    b = lax.axis_index("c")
    s = lax.axis_index("s")
    bb = b * HW
    woff0 = 2 * bb
    woff1 = 2 * bb + HW

    def stage(row, col0, st):
      q = row * W + col0
      pltpu.sync_copy(warp_hbm.at[pl.ds(woff0 + q, P)], wp_vs[0])
      pltpu.sync_copy(warp_hbm.at[pl.ds(woff1 + q, P)], wp_vs[1])
      rowf = row.astype(jnp.float32)
      for g in range(G):
        colf = (col0 + g * L).astype(jnp.float32)
        ii = lax.iota(jnp.int32, L).astype(jnp.float32)
        sl = pl.ds(g * L, L)
        sx = colf + ii + wp_vs[0][sl]
        sy = rowf + wp_vs[1][sl]
        x0i, x0f = _floor(sx)
        y0i, y0f = _floor(sy)
        wx = sx - x0f
        wy = sy - y0f
        bx0 = (x0f >= 0.0) & (x0f <= W - 1.0)
        bx1 = (x0f >= -1.0) & (x0f <= W - 2.0)
        by0 = (y0f >= 0.0) & (y0f <= H - 1.0)
        by1 = (y0f >= -1.0) & (y0f <= H - 2.0)
        ix0 = jnp.clip(x0i, 0, W - 1)
        ix1 = jnp.clip(x0i + 1, 0, W - 1)
        ry0 = bb + jnp.clip(y0i, 0, H - 1) * W
        ry1 = bb + jnp.clip(y0i + 1, 0, H - 1) * W
        idx_vs[st][0][sl] = ry0 + ix0
        idx_vs[st][1][sl] = ry0 + ix1
        idx_vs[st][2][sl] = ry1 + ix0
        idx_vs[st][3][sl] = ry1 + ix1
        zero = jnp.zeros((L,), jnp.float32)
        w_vs[st][0][sl] = jnp.where(bx0 & by0, (1.0 - wx) * (1.0 - wy), zero)
        w_vs[st][1][sl] = jnp.where(bx1 & by0, wx * (1.0 - wy), zero)
        w_vs[st][2][sl] = jnp.where(bx0 & by1, (1.0 - wx) * wy, zero)
        w_vs[st][3][sl] = jnp.where(bx1 & by1, wx * wy, zero)
      for k4 in range(4):
        pltpu.async_copy(xt_hbm.at[idx_vs[st][k4]], row_vs[st][k4], sems[st])

    def drain(st):
      for k4 in range(4):
        pltpu.make_async_copy(xt_hbm.at[idx_vs[st][k4]], row_vs[st][k4],
                              sems[st]).wait()

    def combine_out(row, col0, st):
      @plsc.parallel_loop(0, P, 1, unroll=2)
      def _pix(pi):
        pidx = jnp.full((L,), pi, jnp.int32)
        ws = [plsc.load_gather(w_vs[st][k4], [pidx]) for k4 in range(4)]
        for cb in range(CB):
          sl = pl.ds(cb * L, L)
          acc = ws[0] * row_vs[st][0][pi, sl]
          acc = acc + ws[1] * row_vs[st][1][pi, sl]
          acc = acc + ws[2] * row_vs[st][2][pi, sl]
          acc = acc + ws[3] * row_vs[st][3][pi, sl]
          out_v[pi, sl] = acc

      pltpu.sync_copy(out_v, out_hbm.at[pl.ds(bb + row * W + col0, P)])

    def rc(t):
      r3 = t // CPR
      return s * ROWS_W + r3, (t - r3 * CPR) * P

    r0, c0 = rc(jnp.int32(0))
    stage(r0, c0, 0)

    @pl.loop(0, NCHUNK, step=2)
    def _chunks(t):
      r1, c1 = rc(t + 1)
      stage(r1, c1, 1)
      ra, ca = rc(t)
      drain(0)
      combine_out(ra, ca, 0)

      @pl.when(t + 2 < NCHUNK)
      def _():
        r2, c2 = rc(t + 2)
        stage(r2, c2, 0)

      drain(1)
      combine_out(r1, c1, 1)

  return k(xt, warp)


def kernel(x, warp):
  xt = _tc_transpose_in(x.astype(jnp.float32))
  out_t = _sc_resample(xt, warp.astype(jnp.float32).reshape(2 * NPIX))
  return _tc_transpose_out(out_t)


# speedup vs baseline: 3.3050x; 2.5140x over previous
import dataclasses
import functools

import jax
import jax.numpy as jnp
from jax import lax
from jax.experimental import pallas as pl
from jax.experimental.pallas import tpu as pltpu
from jax.experimental.pallas import tpu_sc as plsc

B, C, H, W = 2, 96, 384, 384
CP = 128
HW = H * W
NPIX = B * HW
NC, NS, L = 2, 16, 16
ROWS_W = H // NS
P = 96
CPR = W // P
NCHUNK = ROWS_W * CPR
G = P // L
CB = C // L
HB = 8


def _floor(v):
  t = v.astype(jnp.int32)
  tf = t.astype(jnp.float32)
  adj = jnp.where(tf > v, 1, 0)
  return t - adj, tf - adj.astype(jnp.float32)


def _tc_transpose_in(x):
  def body(x_ref, o_ref):
    for h in range(HB):
      blk = x_ref[0, :, h, :]
      o_ref[0, h, :, 0:C] = blk.T
      o_ref[0, h, :, C:CP] = jnp.zeros((W, CP - C), jnp.float32)

  return pl.pallas_call(
      body,
      grid=(B, H // HB),
      in_specs=[pl.BlockSpec((1, C, HB, W), lambda b, i: (b, 0, i, 0))],
      out_specs=pl.BlockSpec((1, HB, W, CP), lambda b, i: (b, i, 0, 0)),
      out_shape=jax.ShapeDtypeStruct((B, H, W, CP), jnp.float32),
  )(x)


def _tc_transpose_out(y4):
  def body(y_ref, o_ref):
    for h in range(HB):
      o_ref[0, :, h, :] = y_ref[0, h, :, 0:C].T

  return pl.pallas_call(
      body,
      grid=(B, H // HB),
      in_specs=[pl.BlockSpec((1, HB, W, CP), lambda b, i: (b, i, 0, 0))],
      out_specs=pl.BlockSpec((1, C, HB, W), lambda b, i: (b, 0, i, 0)),
      out_shape=jax.ShapeDtypeStruct((B, C, H, W), jnp.float32),
  )(y4)


def _sc_resample(xt, warp):
  mesh = plsc.VectorSubcoreMesh(core_axis_name="c", subcore_axis_name="s")
  cp = pltpu.CompilerParams()
  if "needs_layout_passes" in pltpu.CompilerParams.__dataclass_fields__:
    cp = dataclasses.replace(cp, needs_layout_passes=False)

  @functools.partial(
      pl.kernel,
      compiler_params=cp,
      out_type=jax.ShapeDtypeStruct((NPIX, CP), jnp.float32),
      mesh=mesh,
      scratch_types=[
          [[pltpu.VMEM((P,), jnp.int32) for _ in range(4)] for _ in range(2)],
          [[pltpu.VMEM((P,), jnp.float32) for _ in range(4)] for _ in range(2)],
          [[pltpu.VMEM((P, CP), jnp.float32) for _ in range(4)]
           for _ in range(2)],
          [pltpu.VMEM((P,), jnp.float32) for _ in range(2)],
          pltpu.VMEM((P, CP), jnp.float32),
          [pltpu.SemaphoreType.DMA for _ in range(2)],
      ],
  )
  def k(xt_hbm, warp_hbm, out_hbm, idx_vs, w_vs, row_vs, wp_vs, out_v, sems):
    b = lax.axis_index("c")
    s = lax.axis_index("s")
    bb = b * HW
    woff0 = 2 * bb
    woff1 = 2 * bb + HW

    def stage(row, col0, st):
      q = row * W + col0
      pltpu.sync_copy(warp_hbm.at[pl.ds(woff0 + q, P)], wp_vs[0])
      pltpu.sync_copy(warp_hbm.at[pl.ds(woff1 + q, P)], wp_vs[1])
      rowf = row.astype(jnp.float32)
      for g in range(G):
        colf = (col0 + g * L).astype(jnp.float32)
        ii = lax.iota(jnp.int32, L).astype(jnp.float32)
        sl = pl.ds(g * L, L)
        sx = colf + ii + wp_vs[0][sl]
        sy = rowf + wp_vs[1][sl]
        x0i, x0f = _floor(sx)
        y0i, y0f = _floor(sy)
        wx = sx - x0f
        wy = sy - y0f
        bx0 = (x0f >= 0.0) & (x0f <= W - 1.0)
        bx1 = (x0f >= -1.0) & (x0f <= W - 2.0)
        by0 = (y0f >= 0.0) & (y0f <= H - 1.0)
        by1 = (y0f >= -1.0) & (y0f <= H - 2.0)
        ix0 = jnp.clip(x0i, 0, W - 1)
        ix1 = jnp.clip(x0i + 1, 0, W - 1)
        ry0 = bb + jnp.clip(y0i, 0, H - 1) * W
        ry1 = bb + jnp.clip(y0i + 1, 0, H - 1) * W
        idx_vs[st][0][sl] = ry0 + ix0
        idx_vs[st][1][sl] = ry0 + ix1
        idx_vs[st][2][sl] = ry1 + ix0
        idx_vs[st][3][sl] = ry1 + ix1
        zero = jnp.zeros((L,), jnp.float32)
        w_vs[st][0][sl] = jnp.where(bx0 & by0, (1.0 - wx) * (1.0 - wy), zero)
        w_vs[st][1][sl] = jnp.where(bx1 & by0, wx * (1.0 - wy), zero)
        w_vs[st][2][sl] = jnp.where(bx0 & by1, (1.0 - wx) * wy, zero)
        w_vs[st][3][sl] = jnp.where(bx1 & by1, wx * wy, zero)
      for k4 in range(4):
        pltpu.async_copy(xt_hbm.at[idx_vs[st][k4]], row_vs[st][k4], sems[st])

    def drain(st):
      for k4 in range(4):
        pltpu.make_async_copy(xt_hbm.at[idx_vs[st][k4]], row_vs[st][k4],
                              sems[st]).wait()

    def combine_out(row, col0, st):
      @plsc.parallel_loop(0, P, 1, unroll=2)
      def _pix(pi):
        pidx = jnp.full((L,), pi, jnp.int32)
        ws = [plsc.load_gather(w_vs[st][k4], [pidx]) for k4 in range(4)]
        for cb in range(CB):
          sl = pl.ds(cb * L, L)
          acc = ws[0] * row_vs[st][0][pi, sl]
          acc = acc + ws[1] * row_vs[st][1][pi, sl]
          acc = acc + ws[2] * row_vs[st][2][pi, sl]
          acc = acc + ws[3] * row_vs[st][3][pi, sl]
          out_v[pi, sl] = acc

      pltpu.sync_copy(out_v, out_hbm.at[pl.ds(bb + row * W + col0, P)])

    def rc(t):
      r4 = t // CPR
      return s * ROWS_W + r4, (t - r4 * CPR) * P

    r0, c0 = rc(jnp.int32(0))
    stage(r0, c0, 0)

    @pl.loop(0, NCHUNK, step=2)
    def _chunks(t):
      r1, c1 = rc(t + 1)
      stage(r1, c1, 1)
      ra, ca = rc(t)
      drain(0)
      combine_out(ra, ca, 0)

      @pl.when(t + 2 < NCHUNK)
      def _():
        r2, c2 = rc(t + 2)
        stage(r2, c2, 0)

      drain(1)
      combine_out(r1, c1, 1)

  return k(xt, warp)


def kernel(x, warp):
  xt4 = _tc_transpose_in(x.astype(jnp.float32))
  out_t = _sc_resample(xt4.reshape(NPIX, CP),
                       warp.astype(jnp.float32).reshape(2 * NPIX))
  return _tc_transpose_out(out_t.reshape(B, H, W, CP))
